# baseline (speedup 1.0000x reference)
import jax
import jax.numpy as jnp
from jax.experimental import pallas as pl


GB = 4


def _mp_kernel(seq_ref, adj_ref, w_ref, ws_ref, b_ref, out_ref):
    for g in range(GB):
        x = seq_ref[g]
        a = (adj_ref[g] != 0).astype(jnp.float32)
        h = jnp.dot(x, w_ref[...], preferred_element_type=jnp.float32)
        agg = jax.lax.dot_general(
            a, h, (((0,), (0,)), ((), ())), preferred_element_type=jnp.float32
        )
        self_term = jnp.dot(x, ws_ref[...], preferred_element_type=jnp.float32)
        out_ref[g] = agg + self_term + b_ref[...]


def kernel(seq, mask, adj_matrix, W, W_self, b):
    B, L, d = seq.shape
    del mask
    b2d = b.reshape(1, d)
    out = pl.pallas_call(
        _mp_kernel,
        grid=(B // GB,),
        in_specs=[
            pl.BlockSpec((GB, L, d), lambda i: (i, 0, 0)),
            pl.BlockSpec((GB, L, L), lambda i: (i, 0, 0)),
            pl.BlockSpec((d, d), lambda i: (0, 0)),
            pl.BlockSpec((d, d), lambda i: (0, 0)),
            pl.BlockSpec((1, d), lambda i: (0, 0)),
        ],
        out_specs=pl.BlockSpec((GB, L, d), lambda i: (i, 0, 0)),
        out_shape=jax.ShapeDtypeStruct((B, L, d), jnp.float32),
    )(seq, adj_matrix, W, W_self, b2d)
    return out


# speedup vs baseline: 1.0607x; 1.0607x over previous
import jax
import jax.numpy as jnp
from jax.experimental import pallas as pl


GB = 2

_CONTRACT_SRC = (((0,), (0,)), ((), ()))


def _mp_kernel(seq_ref, adj_ref, w_ref, ws_ref, b_ref, out_ref):
    for g in range(GB):
        x = seq_ref[g]
        a = (adj_ref[g] != 0).astype(jnp.bfloat16)
        h = jnp.dot(x, w_ref[...], preferred_element_type=jnp.float32)
        h_hi = h.astype(jnp.bfloat16)
        h_lo = (h - h_hi.astype(jnp.float32)).astype(jnp.bfloat16)
        agg = jax.lax.dot_general(
            a, h_hi, _CONTRACT_SRC, preferred_element_type=jnp.float32
        ) + jax.lax.dot_general(
            a, h_lo, _CONTRACT_SRC, preferred_element_type=jnp.float32
        )
        self_term = jnp.dot(x, ws_ref[...], preferred_element_type=jnp.float32)
        out_ref[g] = agg + self_term + b_ref[...]


def kernel(seq, mask, adj_matrix, W, W_self, b):
    B, L, d = seq.shape
    del mask
    b2d = b.reshape(1, d)
    out = pl.pallas_call(
        _mp_kernel,
        grid=(B // GB,),
        in_specs=[
            pl.BlockSpec((GB, L, d), lambda i: (i, 0, 0)),
            pl.BlockSpec((GB, L, L), lambda i: (i, 0, 0)),
            pl.BlockSpec((d, d), lambda i: (0, 0)),
            pl.BlockSpec((d, d), lambda i: (0, 0)),
            pl.BlockSpec((1, d), lambda i: (0, 0)),
        ],
        out_specs=pl.BlockSpec((GB, L, d), lambda i: (i, 0, 0)),
        out_shape=jax.ShapeDtypeStruct((B, L, d), jnp.float32),
    )(seq, adj_matrix, W, W_self, b2d)
    return out
